# baseline (speedup 1.0000x reference)
import functools

import jax
import jax.numpy as jnp
from jax import lax
from jax.experimental import pallas as pl
from jax.experimental.pallas import tpu as pltpu
from jax.experimental.pallas import tpu_sc as plsc


_CHUNK = 128


@functools.cache
def _make_gather(B, d_big, d_small):
    info = plsc.get_sparse_core_info()
    NC, NS = info.num_cores, info.num_subcores
    NW = NC * NS
    assert B % NW == 0
    bpw = B // NW
    assert bpw % _CHUNK == 0
    n_chunks = bpw // _CHUNK
    mesh = plsc.VectorSubcoreMesh(core_axis_name="c", subcore_axis_name="s")
    f32 = jnp.float32

    @functools.partial(
        pl.kernel,
        mesh=mesh,
        compiler_params=pltpu.CompilerParams(use_tc_tiling_on_sc=False),
        out_type=[
            jax.ShapeDtypeStruct((B, d_big), f32),
            jax.ShapeDtypeStruct((B, d_big), f32),
            jax.ShapeDtypeStruct((B, d_small), f32),
            jax.ShapeDtypeStruct((B, d_small), f32),
        ],
        scratch_types=[
            pltpu.VMEM((bpw,), jnp.int32),
            pltpu.VMEM((bpw,), jnp.int32),
            pltpu.VMEM((bpw,), jnp.int32),
            pltpu.VMEM((bpw,), jnp.int32),
            pltpu.VMEM((bpw, d_big), f32),
            pltpu.VMEM((bpw, d_big), f32),
            pltpu.VMEM((bpw, d_small), f32),
            pltpu.VMEM((bpw, d_small), f32),
            pltpu.SemaphoreType.DMA,
        ],
    )
    def gather(u_tab, m_tab, c_tab, d_tab, ui, mi, ci, di,
               ue_out, me_out, ce_out, de_out,
               ui_v, mi_v, ci_v, di_v, ue_v, me_v, ce_v, de_v, sem):
        wid = lax.axis_index("s") * NC + lax.axis_index("c")
        base = wid * bpw
        sl = pl.ds(base, bpw)
        pltpu.sync_copy(ui.at[sl], ui_v)
        pltpu.sync_copy(mi.at[sl], mi_v)
        pltpu.sync_copy(ci.at[sl], ci_v)
        pltpu.sync_copy(di.at[sl], di_v)
        copies = []
        for j in range(n_chunks):
            cs = pl.ds(j * _CHUNK, _CHUNK)
            copies.append(pltpu.async_copy(u_tab.at[ui_v.at[cs]], ue_v.at[cs], sem))
            copies.append(pltpu.async_copy(m_tab.at[mi_v.at[cs]], me_v.at[cs], sem))
            copies.append(pltpu.async_copy(c_tab.at[ci_v.at[cs]], ce_v.at[cs], sem))
            copies.append(pltpu.async_copy(d_tab.at[di_v.at[cs]], de_v.at[cs], sem))
        for c in copies:
            c.wait()
        pltpu.sync_copy(ue_v, ue_out.at[sl])
        pltpu.sync_copy(me_v, me_out.at[sl])
        pltpu.sync_copy(ce_v, ce_out.at[sl])
        pltpu.sync_copy(de_v, de_out.at[sl])

    return gather


def _mlp_body(num_r, ue_r, me_r, ce_r, de_r,
              w1n_r, w1u_r, w1m_r, w1cd_r, b1_r, w2_r, b2_r, w3_r, b3_r,
              out_r):
    f32 = jnp.float32
    cd = jnp.concatenate([ce_r[...], de_r[...]], axis=1)
    h = jnp.dot(num_r[...], w1n_r[...], preferred_element_type=f32)
    h += jnp.dot(ue_r[...], w1u_r[...], preferred_element_type=f32)
    h += jnp.dot(me_r[...], w1m_r[...], preferred_element_type=f32)
    h += jnp.dot(cd, w1cd_r[...], preferred_element_type=f32)
    h = jnp.maximum(h + b1_r[...], 0.0)
    h2 = jnp.dot(h, w2_r[...], preferred_element_type=f32)
    h2 = jnp.maximum(h2 + b2_r[...], 0.0)
    o = jnp.dot(h2, w3_r[...], preferred_element_type=f32)
    out_r[...] = jax.nn.sigmoid(o + b3_r[...])


@functools.cache
def _make_mlp(B, TB, n_num, d_big, d_small, H1, H2):
    grid = (B // TB,)
    d_cd = 2 * d_small

    def row_block(d):
        return pl.BlockSpec((TB, d), lambda i: (i, 0))

    def full(shape):
        return pl.BlockSpec(shape, lambda i: tuple(0 for _ in shape))

    return pl.pallas_call(
        _mlp_body,
        grid=grid,
        in_specs=[
            row_block(n_num), row_block(d_big), row_block(d_big),
            row_block(d_small), row_block(d_small),
            full((n_num, H1)), full((d_big, H1)), full((d_big, H1)),
            full((d_cd, H1)), full((1, H1)),
            full((H1, H2)), full((1, H2)),
            full((H2, 1)), full((1, 1)),
        ],
        out_specs=pl.BlockSpec((TB, 1), lambda i: (i, 0)),
        out_shape=jax.ShapeDtypeStruct((B, 1), jnp.float32),
    )


def kernel(num, user, merchant, country, device,
           user_emb, merchant_emb, country_emb, device_emb,
           W1, b1, W2, b2, W3, b3):
    B, n_num = num.shape
    d_big = user_emb.shape[1]
    d_small = country_emb.shape[1]
    H1 = W1.shape[0]
    H2 = W2.shape[0]

    i32 = jnp.int32
    gather = _make_gather(B, d_big, d_small)
    ue, me, ce, de = gather(
        user_emb, merchant_emb, country_emb, device_emb,
        user.astype(i32), merchant.astype(i32),
        country.astype(i32), device.astype(i32))

    c0 = n_num
    c1 = c0 + d_big
    c2 = c1 + d_big
    w1n = W1[:, :c0].T
    w1u = W1[:, c0:c1].T
    w1m = W1[:, c1:c2].T
    w1cd = W1[:, c2:].T
    mlp = _make_mlp(B, 2048, n_num, d_big, d_small, H1, H2)
    return mlp(num, ue, me, ce, de,
               w1n, w1u, w1m, w1cd, b1.reshape(1, H1),
               W2.T, b2.reshape(1, H2), W3.T, b3.reshape(1, 1))


# speedup vs baseline: 2.0434x; 2.0434x over previous
import functools

import jax
import jax.numpy as jnp
from jax import lax
from jax.experimental import pallas as pl
from jax.experimental.pallas import tpu as pltpu
from jax.experimental.pallas import tpu_sc as plsc


_G = 16


@functools.cache
def _make_gather(B, d_big, d_small, cpad, dpad):
    info = plsc.get_sparse_core_info()
    NC, NS = info.num_cores, info.num_subcores
    NW = NC * NS
    assert B % NW == 0
    bpw = B // NW
    assert bpw % _G == 0
    ngroups = bpw // _G
    mesh = plsc.VectorSubcoreMesh(core_axis_name="c", subcore_axis_name="s")
    f32 = jnp.float32
    i32 = jnp.int32

    @functools.partial(
        pl.kernel,
        mesh=mesh,
        compiler_params=pltpu.CompilerParams(
            use_tc_tiling_on_sc=True, needs_layout_passes=False),
        out_type=jax.ShapeDtypeStruct((B, 128), f32),
        scratch_types=[
            pltpu.VMEM((bpw,), i32),
            pltpu.VMEM((bpw,), i32),
            pltpu.VMEM((bpw,), i32),
            pltpu.VMEM((bpw,), i32),
            pltpu.VMEM((bpw,), i32),
            pltpu.VMEM((bpw,), i32),
            pltpu.VMEM((_G, d_big, 128), f32),
            pltpu.VMEM((d_small, cpad), f32),
            pltpu.VMEM((d_small, dpad), f32),
            pltpu.VMEM((_G, 128), f32),
            pltpu.SemaphoreType.DMA,
            pltpu.SemaphoreType.DMA,
        ],
    )
    def gather(userT, merchT, ctryT, devT, uq, ur, mq, mr, ci, di,
               xg_out,
               uq_v, ur_v, mq_v, mr_v, ci_v, di_v,
               stage_v, ctry_v, dev_v, xga_v, sem, sem2):
        wid = lax.axis_index("s") * NC + lax.axis_index("c")
        base = wid * bpw
        sl = pl.ds(base, bpw)
        pltpu.sync_copy(uq.at[sl], uq_v)
        pltpu.sync_copy(ur.at[sl], ur_v)
        pltpu.sync_copy(mq.at[sl], mq_v)
        pltpu.sync_copy(mr.at[sl], mr_v)
        pltpu.sync_copy(ci.at[sl], ci_v)
        pltpu.sync_copy(di.at[sl], di_v)
        pltpu.sync_copy(ctryT, ctry_v)
        pltpu.sync_copy(devT, dev_v)

        iota16 = lax.iota(i32, 16)

        def group(g, _):
            i0 = g * _G

            def do_big(q_ref, r_v, colbase):
                q_vec = q_ref[pl.ds(i0, _G)]
                cps = []
                for s in range(_G):
                    q = q_vec[s]
                    col = pl.multiple_of(q * 128, 128)
                    cps.append(pltpu.async_copy(
                        tabT.at[:, pl.ds(col, 128)], stage_v.at[s], sem))
                for c in cps:
                    c.wait()
                rv = r_v[pl.ds(i0, 16)]
                for f in range(d_big):
                    v = plsc.load_gather(
                        stage_v, [iota16, jnp.full((16,), f, i32), rv])
                    plsc.store_scatter(
                        xga_v, [iota16, jnp.full((16,), colbase + f, i32)], v)

            tabT = userT
            do_big(uq_v, ur_v, 0)
            tabT = merchT
            do_big(mq_v, mr_v, d_big)

            cv = ci_v[pl.ds(i0, 16)]
            dv = di_v[pl.ds(i0, 16)]
            for f in range(d_small):
                fvec = jnp.full((16,), f, i32)
                v = plsc.load_gather(ctry_v, [fvec, cv])
                plsc.store_scatter(
                    xga_v, [iota16, jnp.full((16,), 2 * d_big + f, i32)], v)
                v = plsc.load_gather(dev_v, [fvec, dv])
                plsc.store_scatter(
                    xga_v,
                    [iota16, jnp.full((16,), 2 * d_big + d_small + f, i32)], v)

            pltpu.sync_copy(xga_v, xg_out.at[pl.ds(base + i0, _G)])
            return 0

        lax.fori_loop(0, ngroups, group, 0)

    return gather


def _mlp_body(numT_r, xg_r, w1n_r, w1x_r, b1_r, w2_r, b2_r, w3_r, b3_r, out_r):
    f32 = jnp.float32
    dn = (((0,), (0,)), ((), ()))
    h = lax.dot_general(numT_r[...], w1n_r[...], dn, preferred_element_type=f32)
    h += jnp.dot(xg_r[:, :80], w1x_r[...], preferred_element_type=f32)
    h = jnp.maximum(h + b1_r[...], 0.0)
    h2 = jnp.dot(h, w2_r[...], preferred_element_type=f32)
    h2 = jnp.maximum(h2 + b2_r[...], 0.0)
    o = jnp.dot(h2, w3_r[...], preferred_element_type=f32)
    out_r[...] = jax.nn.sigmoid(o + b3_r[...])


@functools.cache
def _make_mlp(B, TB, n_num, H1, H2):
    grid = (B // TB,)

    def full(shape):
        return pl.BlockSpec(shape, lambda i: tuple(0 for _ in shape))

    return pl.pallas_call(
        _mlp_body,
        grid=grid,
        in_specs=[
            pl.BlockSpec((n_num, TB), lambda i: (0, i)),
            pl.BlockSpec((TB, 128), lambda i: (i, 0)),
            full((n_num, H1)), full((80, H1)), full((1, H1)),
            full((H1, H2)), full((1, H2)),
            full((H2, 1)), full((1, 1)),
        ],
        out_specs=pl.BlockSpec((TB, 1), lambda i: (i, 0)),
        out_shape=jax.ShapeDtypeStruct((B, 1), jnp.float32),
    )


def kernel(num, user, merchant, country, device,
           user_emb, merchant_emb, country_emb, device_emb,
           W1, b1, W2, b2, W3, b3):
    B, n_num = num.shape
    d_big = user_emb.shape[1]
    d_small = country_emb.shape[1]
    H1 = W1.shape[0]
    H2 = W2.shape[0]

    i32 = jnp.int32
    user = user.astype(i32)
    merchant = merchant.astype(i32)

    def pad128(n):
        return -(-n // 128) * 128

    cpad = pad128(country_emb.shape[0])
    dpad = pad128(device_emb.shape[0])
    ctryT = jnp.pad(country_emb, ((0, cpad - country_emb.shape[0]), (0, 0))).T
    devT = jnp.pad(device_emb, ((0, dpad - device_emb.shape[0]), (0, 0))).T

    gather = _make_gather(B, d_big, d_small, cpad, dpad)
    xg = gather(user_emb.T, merchant_emb.T, ctryT, devT,
                user // 128, user % 128, merchant // 128, merchant % 128,
                country.astype(i32), device.astype(i32))

    c0 = n_num
    w1n = W1[:, :c0].T
    w1x = W1[:, c0:].T
    mlp = _make_mlp(B, 2048, n_num, H1, H2)
    return mlp(num.T, xg,
               w1n, w1x, b1.reshape(1, H1),
               W2.T, b2.reshape(1, H2), W3.T, b3.reshape(1, 1))


# speedup vs baseline: 2.1463x; 1.0503x over previous
import functools

import jax
import jax.numpy as jnp
from jax import lax
from jax.experimental import pallas as pl
from jax.experimental.pallas import tpu as pltpu
from jax.experimental.pallas import tpu_sc as plsc


_G = 8


@functools.cache
def _make_gather(B, d_big, d_small, cpad, dpad):
    info = plsc.get_sparse_core_info()
    NC, NS = info.num_cores, info.num_subcores
    NW = NC * NS
    assert B % NW == 0
    bpw = B // NW
    assert bpw % _G == 0
    ngroups = bpw // _G
    mesh = plsc.VectorSubcoreMesh(core_axis_name="c", subcore_axis_name="s")
    f32 = jnp.float32
    i32 = jnp.int32

    @functools.partial(
        pl.kernel,
        mesh=mesh,
        compiler_params=pltpu.CompilerParams(
            use_tc_tiling_on_sc=True, needs_layout_passes=False),
        out_type=jax.ShapeDtypeStruct((B, 128), f32),
        scratch_types=[
            pltpu.VMEM((bpw + 16,), i32),
            pltpu.VMEM((bpw,), i32),
            pltpu.VMEM((bpw + 16,), i32),
            pltpu.VMEM((bpw,), i32),
            pltpu.VMEM((bpw,), i32),
            pltpu.VMEM((bpw,), i32),
            pltpu.VMEM((_G, d_big, 128), f32),
            pltpu.VMEM((_G, d_big, 128), f32),
            pltpu.VMEM((d_small, cpad), f32),
            pltpu.VMEM((d_small, dpad), f32),
            pltpu.VMEM((_G, 128), f32),
            pltpu.SemaphoreType.DMA,
            pltpu.SemaphoreType.DMA,
        ],
    )
    def gather(userT, merchT, ctryT, devT, uq, ur, mq, mr, ci, di,
               xg_out,
               uq_v, ur_v, mq_v, mr_v, ci_v, di_v,
               ustage, mstage, ctry_v, dev_v, xga_v, semu, semm):
        wid = lax.axis_index("s") * NC + lax.axis_index("c")
        base = wid * bpw
        sl = pl.ds(base, bpw)
        pltpu.sync_copy(uq.at[sl], uq_v.at[pl.ds(0, bpw)])
        pltpu.sync_copy(ur.at[sl], ur_v)
        pltpu.sync_copy(mq.at[sl], mq_v.at[pl.ds(0, bpw)])
        pltpu.sync_copy(mr.at[sl], mr_v)
        pltpu.sync_copy(ci.at[sl], ci_v)
        pltpu.sync_copy(di.at[sl], di_v)
        pltpu.sync_copy(ctryT, ctry_v)
        pltpu.sync_copy(devT, dev_v)

        iota16 = lax.iota(i32, 16)
        slot8 = iota16 % _G
        fpair = iota16 // _G

        def fire(tabT, q_ref, stage, sem, g):
            q_vec = q_ref[pl.ds(g * _G, 16)]
            for s in range(_G):
                col = pl.multiple_of(q_vec[s] * 128, 128)
                pltpu.async_copy(tabT.at[:, pl.ds(col, 128)], stage.at[s], sem)

        def drain(tabT, stage, sem):
            for s in range(_G):
                pltpu.make_async_copy(
                    tabT.at[:, pl.ds(0, 128)], stage.at[s], sem).wait()

        def extract(stage, r_v, colbase, g):
            rv = plsc.load_gather(r_v, [g * _G + slot8])
            for f in range(0, d_big, 2):
                v = plsc.load_gather(stage, [slot8, fpair + f, rv])
                plsc.store_scatter(
                    xga_v, [slot8, fpair + (colbase + f)], v)

        def extract_small(tab_v, idx_v, colbase, g):
            iv = plsc.load_gather(idx_v, [g * _G + slot8])
            for f in range(0, d_small, 2):
                v = plsc.load_gather(tab_v, [fpair + f, iv])
                plsc.store_scatter(
                    xga_v, [slot8, fpair + (colbase + f)], v)

        fire(userT, uq_v, ustage, semu, 0)

        def group(g, _):
            fire(merchT, mq_v, mstage, semm, g)
            drain(userT, ustage, semu)
            extract(ustage, ur_v, 0, g)

            @pl.when(g < ngroups - 1)
            def _():
                fire(userT, uq_v, ustage, semu, g + 1)

            drain(merchT, mstage, semm)
            extract(mstage, mr_v, d_big, g)
            extract_small(ctry_v, ci_v, 2 * d_big, g)
            extract_small(dev_v, di_v, 2 * d_big + d_small, g)
            pltpu.sync_copy(xga_v, xg_out.at[pl.ds(base + g * _G, _G)])
            return 0

        lax.fori_loop(0, ngroups, group, 0)

    return gather


def _mlp_body(numT_r, xg_r, w1n_r, w1x_r, b1_r, w2_r, b2_r, w3_r, b3_r, out_r):
    f32 = jnp.float32
    dn = (((0,), (0,)), ((), ()))
    h = lax.dot_general(numT_r[...], w1n_r[...], dn, preferred_element_type=f32)
    h += jnp.dot(xg_r[:, :80], w1x_r[...], preferred_element_type=f32)
    h = jnp.maximum(h + b1_r[...], 0.0)
    h2 = jnp.dot(h, w2_r[...], preferred_element_type=f32)
    h2 = jnp.maximum(h2 + b2_r[...], 0.0)
    o = jnp.dot(h2, w3_r[...], preferred_element_type=f32)
    out_r[...] = jax.nn.sigmoid(o + b3_r[...])


@functools.cache
def _make_mlp(B, TB, n_num, H1, H2):
    grid = (B // TB,)

    def full(shape):
        return pl.BlockSpec(shape, lambda i: tuple(0 for _ in shape))

    return pl.pallas_call(
        _mlp_body,
        grid=grid,
        in_specs=[
            pl.BlockSpec((n_num, TB), lambda i: (0, i)),
            pl.BlockSpec((TB, 128), lambda i: (i, 0)),
            full((n_num, H1)), full((80, H1)), full((1, H1)),
            full((H1, H2)), full((1, H2)),
            full((H2, 1)), full((1, 1)),
        ],
        out_specs=pl.BlockSpec((TB, 1), lambda i: (i, 0)),
        out_shape=jax.ShapeDtypeStruct((B, 1), jnp.float32),
    )


def kernel(num, user, merchant, country, device,
           user_emb, merchant_emb, country_emb, device_emb,
           W1, b1, W2, b2, W3, b3):
    B, n_num = num.shape
    d_big = user_emb.shape[1]
    d_small = country_emb.shape[1]
    H1 = W1.shape[0]
    H2 = W2.shape[0]

    i32 = jnp.int32
    user = user.astype(i32)
    merchant = merchant.astype(i32)

    def pad128(n):
        return -(-n // 128) * 128

    cpad = pad128(country_emb.shape[0])
    dpad = pad128(device_emb.shape[0])
    ctryT = jnp.pad(country_emb, ((0, cpad - country_emb.shape[0]), (0, 0))).T
    devT = jnp.pad(device_emb, ((0, dpad - device_emb.shape[0]), (0, 0))).T

    gather = _make_gather(B, d_big, d_small, cpad, dpad)
    xg = gather(user_emb.T, merchant_emb.T, ctryT, devT,
                user // 128, user % 128, merchant // 128, merchant % 128,
                country.astype(i32), device.astype(i32))

    c0 = n_num
    w1n = W1[:, :c0].T
    w1x = W1[:, c0:].T
    mlp = _make_mlp(B, 2048, n_num, H1, H2)
    return mlp(num.T, xg,
               w1n, w1x, b1.reshape(1, H1),
               W2.T, b2.reshape(1, H2), W3.T, b3.reshape(1, 1))
